# baseline (speedup 1.0000x reference)
import functools

import jax
import jax.numpy as jnp
from jax import lax
from jax.experimental import pallas as pl
from jax.experimental.pallas import tpu as pltpu
from jax.experimental.pallas import tpu_sc as plsc

N_CLS = 1000
DIM = 512
N_CTX = 16
SEQ = 77
PREF_W = DIM
CTX_W = N_CTX * DIM
SUF_W = (SEQ - 1 - N_CTX) * DIM
ROW_W = SEQ * DIM
TOK_W = 128
B = 256

NW = 32
ROWS_PER_W = (2 * B) // NW
CHUNK = 2
NCHUNK = ROWS_PER_W // CHUNK


def _sc_body(cls_hbm, cls2_hbm, pref_neg, ctx_neg, suf_neg, pref_pos, ctx_pos,
             suf_pos, tok_hbm, out_hbm, tokout_hbm,
             idx_v, idx2_v, idxtok_v, pref_v, ctx_v, suf_v, tok_v, sem_g, sem_s):
    nc = 2
    wid = lax.axis_index("s") * nc + lax.axis_index("c")
    half = wid // 16
    j = wid % 16
    out_base = wid * ROWS_PER_W
    idx_base = j * ROWS_PER_W

    pltpu.sync_copy(cls_hbm.at[pl.ds(idx_base, ROWS_PER_W)], idx_v)
    pltpu.sync_copy(cls2_hbm.at[j], idx2_v)
    idxtok_v[...] = idx_v[...] + half * N_CLS

    pltpu.async_copy(tok_hbm.at[idxtok_v], tok_v, sem_g).wait()
    pltpu.sync_copy(tok_v, tokout_hbm.at[pl.ds(out_base, ROWS_PER_W)])

    def do_half(pref_t, ctx_t, suf_t):
        for it in range(NCHUNK):
            gb = out_base + it * CHUNK
            i2 = idx2_v.at[it]
            g1 = pltpu.async_copy(pref_t.at[i2], pref_v, sem_g)
            g2 = pltpu.async_copy(ctx_t.at[i2], ctx_v, sem_g)
            g3 = pltpu.async_copy(suf_t.at[i2], suf_v, sem_g)
            g1.wait()
            g2.wait()
            g3.wait()
            s1 = pltpu.async_copy(
                pref_v, out_hbm.at[pl.ds(gb, CHUNK), pl.ds(0, PREF_W)], sem_s)
            s2 = pltpu.async_copy(
                ctx_v, out_hbm.at[pl.ds(gb, CHUNK), pl.ds(PREF_W, CTX_W)], sem_s)
            s3 = pltpu.async_copy(
                suf_v, out_hbm.at[pl.ds(gb, CHUNK), pl.ds(PREF_W + CTX_W, SUF_W)],
                sem_s)
            s1.wait()
            s2.wait()
            s3.wait()

    @pl.when(half == 0)
    def _():
        do_half(pref_neg, ctx_neg, suf_neg)

    @pl.when(half == 1)
    def _():
        do_half(pref_pos, ctx_pos, suf_pos)


def kernel(cls_id, ctx_pos, ctx_neg, token_prefix_pos, token_suffix_pos,
           token_prefix_neg, token_suffix_neg, tokenized_prompts):
    ctx_pos2 = ctx_pos.reshape(N_CLS, CTX_W)
    ctx_neg2 = ctx_neg.reshape(N_CLS, CTX_W)
    pref_pos2 = token_prefix_pos.reshape(N_CLS, PREF_W)
    pref_neg2 = token_prefix_neg.reshape(N_CLS, PREF_W)
    suf_pos2 = token_suffix_pos.reshape(N_CLS, SUF_W)
    suf_neg2 = token_suffix_neg.reshape(N_CLS, SUF_W)

    mesh = plsc.VectorSubcoreMesh(core_axis_name="c", subcore_axis_name="s")
    run = functools.partial(
        pl.kernel,
        mesh=mesh,
        out_type=(
            jax.ShapeDtypeStruct((2 * B, ROW_W), jnp.float32),
            jax.ShapeDtypeStruct((2 * B, TOK_W), jnp.int32),
        ),
        scratch_types=[
            pltpu.VMEM((ROWS_PER_W,), jnp.int32),
            pltpu.VMEM((NCHUNK, CHUNK), jnp.int32),
            pltpu.VMEM((ROWS_PER_W,), jnp.int32),
            pltpu.VMEM((CHUNK, PREF_W), jnp.float32),
            pltpu.VMEM((CHUNK, CTX_W), jnp.float32),
            pltpu.VMEM((CHUNK, SUF_W), jnp.float32),
            pltpu.VMEM((ROWS_PER_W, TOK_W), jnp.int32),
            pltpu.SemaphoreType.DMA,
            pltpu.SemaphoreType.DMA,
        ],
    )(_sc_body)

    cls2 = cls_id.reshape(16, NCHUNK, CHUNK)
    tok_padded = jnp.pad(tokenized_prompts, ((0, 0), (0, TOK_W - SEQ)))
    prompts_flat, tokenized_padded = run(
        cls_id, cls2, pref_neg2, ctx_neg2, suf_neg2, pref_pos2, ctx_pos2,
        suf_pos2, tok_padded)
    return prompts_flat.reshape(2 * B, SEQ, DIM), tokenized_padded[:, :SEQ]


# speedup vs baseline: 1.0089x; 1.0089x over previous
import functools

import jax
import jax.numpy as jnp
from jax import lax
from jax.experimental import pallas as pl
from jax.experimental.pallas import tpu as pltpu
from jax.experimental.pallas import tpu_sc as plsc

N_CLS = 1000
DIM = 512
N_CTX = 16
SEQ = 77
PREF_W = DIM
CTX_W = N_CTX * DIM
SUF_W = (SEQ - 1 - N_CTX) * DIM
ROW_W = SEQ * DIM
TOK_W = 128
B = 256

NW = 32
ROWS_PER_W = (2 * B) // NW
NBUF = 3


def _sc_body(cls_hbm, cls2_hbm, pref_neg, ctx_neg, suf_neg, pref_pos, ctx_pos,
             suf_pos, tok_hbm, out_hbm, tokout_hbm,
             idx_v, idx2_v, idxtok_v,
             pref_v0, pref_v1, pref_v2, ctx_v0, ctx_v1, ctx_v2,
             suf_v0, suf_v1, suf_v2, tok_v,
             sem_t, sem_g0, sem_g1, sem_g2, sem_s0, sem_s1, sem_s2):
    pref_bufs = [pref_v0, pref_v1, pref_v2]
    ctx_bufs = [ctx_v0, ctx_v1, ctx_v2]
    suf_bufs = [suf_v0, suf_v1, suf_v2]
    sems_g = [sem_g0, sem_g1, sem_g2]
    sems_s = [sem_s0, sem_s1, sem_s2]
    nc = 2
    wid = lax.axis_index("s") * nc + lax.axis_index("c")
    half = wid // 16
    j = wid % 16
    out_base = wid * ROWS_PER_W
    idx_base = j * ROWS_PER_W

    pltpu.sync_copy(cls_hbm.at[pl.ds(idx_base, ROWS_PER_W)], idx_v)
    pltpu.sync_copy(cls2_hbm.at[j], idx2_v)
    idxtok_v[...] = idx_v[...] + half * N_CLS

    pltpu.async_copy(tok_hbm.at[idxtok_v], tok_v, sem_t).wait()
    pltpu.sync_copy(tok_v, tokout_hbm.at[pl.ds(out_base, ROWS_PER_W)])

    def do_half(pref_t, ctx_t, suf_t):
        def fire(i):
            b = i % NBUF
            ii = idx2_v.at[i]
            return (
                pltpu.async_copy(pref_t.at[ii], pref_bufs[b], sems_g[b]),
                pltpu.async_copy(ctx_t.at[ii], ctx_bufs[b], sems_g[b]),
                pltpu.async_copy(suf_t.at[ii], suf_bufs[b], sems_g[b]),
            )

        def store(i):
            b = i % NBUF
            gb = pl.ds(out_base + i, 1)
            return (
                pltpu.async_copy(pref_bufs[b],
                                 out_hbm.at[gb, pl.ds(0, PREF_W)], sems_s[b]),
                pltpu.async_copy(ctx_bufs[b],
                                 out_hbm.at[gb, pl.ds(PREF_W, CTX_W)],
                                 sems_s[b]),
                pltpu.async_copy(suf_bufs[b],
                                 out_hbm.at[gb, pl.ds(PREF_W + CTX_W, SUF_W)],
                                 sems_s[b]),
            )

        gh = {}
        sh = {}
        for i in range(NBUF):
            gh[i] = fire(i)
        for i in range(ROWS_PER_W):
            for h in gh[i]:
                h.wait()
            sh[i] = store(i)
            if i + NBUF < ROWS_PER_W:
                for h in sh[i]:
                    h.wait()
                gh[i + NBUF] = fire(i + NBUF)
        for i in range(ROWS_PER_W - NBUF, ROWS_PER_W):
            for h in sh[i]:
                h.wait()

    @pl.when(half == 0)
    def _():
        do_half(pref_neg, ctx_neg, suf_neg)

    @pl.when(half == 1)
    def _():
        do_half(pref_pos, ctx_pos, suf_pos)


def kernel(cls_id, ctx_pos, ctx_neg, token_prefix_pos, token_suffix_pos,
           token_prefix_neg, token_suffix_neg, tokenized_prompts):
    ctx_pos2 = ctx_pos.reshape(N_CLS, CTX_W)
    ctx_neg2 = ctx_neg.reshape(N_CLS, CTX_W)
    pref_pos2 = token_prefix_pos.reshape(N_CLS, PREF_W)
    pref_neg2 = token_prefix_neg.reshape(N_CLS, PREF_W)
    suf_pos2 = token_suffix_pos.reshape(N_CLS, SUF_W)
    suf_neg2 = token_suffix_neg.reshape(N_CLS, SUF_W)

    mesh = plsc.VectorSubcoreMesh(core_axis_name="c", subcore_axis_name="s")
    run = functools.partial(
        pl.kernel,
        mesh=mesh,
        out_type=(
            jax.ShapeDtypeStruct((2 * B, ROW_W), jnp.float32),
            jax.ShapeDtypeStruct((2 * B, TOK_W), jnp.int32),
        ),
        scratch_types=[
            pltpu.VMEM((ROWS_PER_W,), jnp.int32),
            pltpu.VMEM((ROWS_PER_W, 1), jnp.int32),
            pltpu.VMEM((ROWS_PER_W,), jnp.int32),
            pltpu.VMEM((1, PREF_W), jnp.float32),
            pltpu.VMEM((1, PREF_W), jnp.float32),
            pltpu.VMEM((1, PREF_W), jnp.float32),
            pltpu.VMEM((1, CTX_W), jnp.float32),
            pltpu.VMEM((1, CTX_W), jnp.float32),
            pltpu.VMEM((1, CTX_W), jnp.float32),
            pltpu.VMEM((1, SUF_W), jnp.float32),
            pltpu.VMEM((1, SUF_W), jnp.float32),
            pltpu.VMEM((1, SUF_W), jnp.float32),
            pltpu.VMEM((ROWS_PER_W, TOK_W), jnp.int32),
            pltpu.SemaphoreType.DMA,
            pltpu.SemaphoreType.DMA,
            pltpu.SemaphoreType.DMA,
            pltpu.SemaphoreType.DMA,
            pltpu.SemaphoreType.DMA,
            pltpu.SemaphoreType.DMA,
            pltpu.SemaphoreType.DMA,
        ],
    )(_sc_body)

    cls2 = cls_id.reshape(16, ROWS_PER_W, 1)
    tok_padded = jnp.pad(tokenized_prompts, ((0, 0), (0, TOK_W - SEQ)))
    prompts_flat, tokenized_padded = run(
        cls_id, cls2, pref_neg2, ctx_neg2, suf_neg2, pref_pos2, ctx_pos2,
        suf_pos2, tok_padded)
    return prompts_flat.reshape(2 * B, SEQ, DIM), tokenized_padded[:, :SEQ]


# speedup vs baseline: 1.3610x; 1.3491x over previous
import jax
import jax.numpy as jnp
from jax.experimental import pallas as pl
from jax.experimental.pallas import tpu as pltpu

N_CLS = 1000
DIM = 512
N_CTX = 16
SEQ = 77
SUF_L = SEQ - 1 - N_CTX
B = 256


def _tc_body(cls_ref, ctx_n, ctx_p, pref_n, pref_p, suf_n, suf_p,
             tok_n, tok_p, out_ref, tokout_ref):
    out_ref[0, 0, pl.ds(0, 1)] = pref_n[0]
    out_ref[0, 0, pl.ds(1, N_CTX)] = ctx_n[0]
    out_ref[0, 0, pl.ds(1 + N_CTX, SUF_L)] = suf_n[0]
    out_ref[1, 0, pl.ds(0, 1)] = pref_p[0]
    out_ref[1, 0, pl.ds(1, N_CTX)] = ctx_p[0]
    out_ref[1, 0, pl.ds(1 + N_CTX, SUF_L)] = suf_p[0]
    tokout_ref[0, 0, 0] = tok_n[0, 0]
    tokout_ref[1, 0, 0] = tok_p[0, 0]


def kernel(cls_id, ctx_pos, ctx_neg, token_prefix_pos, token_suffix_pos,
           token_prefix_neg, token_suffix_neg, tokenized_prompts):
    def im_cls(b, cls):
        return (cls[b], 0, 0)

    def im_tok_n(b, cls):
        return (cls[b], 0, 0)

    def im_tok_p(b, cls):
        return (N_CLS + cls[b], 0, 0)

    def im_out(b, cls):
        return (0, b, 0, 0)

    grid_spec = pltpu.PrefetchScalarGridSpec(
        num_scalar_prefetch=1,
        grid=(B,),
        in_specs=[
            pl.BlockSpec((1, N_CTX, DIM), im_cls),
            pl.BlockSpec((1, N_CTX, DIM), im_cls),
            pl.BlockSpec((1, 1, DIM), im_cls),
            pl.BlockSpec((1, 1, DIM), im_cls),
            pl.BlockSpec((1, SUF_L, DIM), im_cls),
            pl.BlockSpec((1, SUF_L, DIM), im_cls),
            pl.BlockSpec((1, 1, SEQ), im_tok_n),
            pl.BlockSpec((1, 1, SEQ), im_tok_p),
        ],
        out_specs=[
            pl.BlockSpec((2, 1, SEQ, DIM), im_out),
            pl.BlockSpec((2, 1, 1, SEQ), lambda b, cls: (0, b, 0, 0)),
        ],
    )

    prompts4, tok4 = pl.pallas_call(
        _tc_body,
        grid_spec=grid_spec,
        out_shape=(
            jax.ShapeDtypeStruct((2, B, SEQ, DIM), jnp.float32),
            jax.ShapeDtypeStruct((2, B, 1, SEQ), jnp.int32),
        ),
        compiler_params=pltpu.CompilerParams(
            dimension_semantics=("arbitrary",)),
    )(cls_id, ctx_neg, ctx_pos, token_prefix_neg, token_prefix_pos,
      token_suffix_neg, token_suffix_pos,
      tokenized_prompts.reshape(2 * N_CLS, 1, SEQ),
      tokenized_prompts.reshape(2 * N_CLS, 1, SEQ))

    return prompts4.reshape(2 * B, SEQ, DIM), tok4.reshape(2 * B, SEQ)


# speedup vs baseline: 1.7868x; 1.3128x over previous
import jax
import jax.numpy as jnp
from jax import lax
from jax.experimental import pallas as pl
from jax.experimental.pallas import tpu as pltpu

N_CLS = 1000
DIM = 512
N_CTX = 16
SEQ = 77
SUF_L = SEQ - 1 - N_CTX
B = 256

K = 8
NSTEP = B // K


def _body(cls_ref, ctx_n, ctx_p, pref_n, pref_p, suf_n, suf_p, tok3,
          out_hbm, tokout_hbm,
          pn_v, pp_v, cn_v, cp_v, sn_v, sp_v, tn_v, tp_v, o_v, ot_v,
          sem_g, sem_s):
    i = pl.program_id(0)

    def fire(k, b):
        c = cls_ref[b]
        g = c // 8
        sem = sem_g.at[k]
        pltpu.make_async_copy(pref_n.at[pl.ds(c, 1)], pn_v.at[k], sem).start()
        pltpu.make_async_copy(pref_p.at[pl.ds(c, 1)], pp_v.at[k], sem).start()
        pltpu.make_async_copy(ctx_n.at[pl.ds(c, 1)], cn_v.at[k], sem).start()
        pltpu.make_async_copy(ctx_p.at[pl.ds(c, 1)], cp_v.at[k], sem).start()
        pltpu.make_async_copy(suf_n.at[pl.ds(c, 1)], sn_v.at[k], sem).start()
        pltpu.make_async_copy(suf_p.at[pl.ds(c, 1)], sp_v.at[k], sem).start()
        pltpu.make_async_copy(tok3.at[pl.ds(g, 1)], tn_v.at[k], sem).start()
        pltpu.make_async_copy(tok3.at[pl.ds(125 + g, 1)], tp_v.at[k],
                              sem).start()

    def wait_gathers(k):
        sem = sem_g.at[k]
        pltpu.make_async_copy(pref_n.at[pl.ds(0, 1)], pn_v.at[k], sem).wait()
        pltpu.make_async_copy(pref_p.at[pl.ds(0, 1)], pp_v.at[k], sem).wait()
        pltpu.make_async_copy(ctx_n.at[pl.ds(0, 1)], cn_v.at[k], sem).wait()
        pltpu.make_async_copy(ctx_p.at[pl.ds(0, 1)], cp_v.at[k], sem).wait()
        pltpu.make_async_copy(suf_n.at[pl.ds(0, 1)], sn_v.at[k], sem).wait()
        pltpu.make_async_copy(suf_p.at[pl.ds(0, 1)], sp_v.at[k], sem).wait()
        pltpu.make_async_copy(tok3.at[pl.ds(0, 1)], tn_v.at[k], sem).wait()
        pltpu.make_async_copy(tok3.at[pl.ds(0, 1)], tp_v.at[k], sem).wait()

    def fire_stores(k, b):
        sem = sem_s.at[k]
        pltpu.make_async_copy(o_v.at[k], out_hbm.at[:, pl.ds(b, 1)],
                              sem).start()
        pltpu.make_async_copy(ot_v.at[k], tokout_hbm.at[:, pl.ds(b, 1)],
                              sem).start()

    def wait_stores(k, b):
        sem = sem_s.at[k]
        pltpu.make_async_copy(o_v.at[k], out_hbm.at[:, pl.ds(b, 1)],
                              sem).wait()
        pltpu.make_async_copy(ot_v.at[k], tokout_hbm.at[:, pl.ds(b, 1)],
                              sem).wait()

    @pl.when(i == 0)
    def _():
        for k in range(K):
            fire(k, k)

    for k in range(K):
        b = i * K + k
        wait_gathers(k)

        @pl.when(i > 0)
        def _(k=k, b=b):
            wait_stores(k, b)

        o_v[k, 0, 0, pl.ds(0, 1)] = pn_v[k, 0]
        o_v[k, 0, 0, pl.ds(1, N_CTX)] = cn_v[k, 0]
        o_v[k, 0, 0, pl.ds(1 + N_CTX, SUF_L)] = sn_v[k, 0]
        o_v[k, 1, 0, pl.ds(0, 1)] = pp_v[k, 0]
        o_v[k, 1, 0, pl.ds(1, N_CTX)] = cp_v[k, 0]
        o_v[k, 1, 0, pl.ds(1 + N_CTX, SUF_L)] = sp_v[k, 0]

        c = cls_ref[b]
        r = lax.rem(c, 8)
        rows = lax.broadcasted_iota(jnp.int32, (8, SEQ), 0)
        tn = jnp.sum(jnp.where(rows == r, tn_v[k, 0], 0), axis=0)
        tp = jnp.sum(jnp.where(rows == r, tp_v[k, 0], 0), axis=0)
        ot_v[k, 0, 0] = tn.reshape(1, SEQ)
        ot_v[k, 1, 0] = tp.reshape(1, SEQ)

        fire_stores(k, b)

        @pl.when(i + 1 < NSTEP)
        def _(k=k, b=b):
            fire(k, b + K)

    @pl.when(i == NSTEP - 1)
    def _():
        for k in range(K):
            wait_stores(k, i * K + k)


def kernel(cls_id, ctx_pos, ctx_neg, token_prefix_pos, token_suffix_pos,
           token_prefix_neg, token_suffix_neg, tokenized_prompts):
    grid_spec = pltpu.PrefetchScalarGridSpec(
        num_scalar_prefetch=1,
        grid=(NSTEP,),
        in_specs=[pl.BlockSpec(memory_space=pl.ANY)] * 7,
        out_specs=[
            pl.BlockSpec(memory_space=pl.ANY),
            pl.BlockSpec(memory_space=pl.ANY),
        ],
        scratch_shapes=[
            pltpu.VMEM((K, 1, 1, DIM), jnp.float32),
            pltpu.VMEM((K, 1, 1, DIM), jnp.float32),
            pltpu.VMEM((K, 1, N_CTX, DIM), jnp.float32),
            pltpu.VMEM((K, 1, N_CTX, DIM), jnp.float32),
            pltpu.VMEM((K, 1, SUF_L, DIM), jnp.float32),
            pltpu.VMEM((K, 1, SUF_L, DIM), jnp.float32),
            pltpu.VMEM((K, 1, 8, SEQ), jnp.int32),
            pltpu.VMEM((K, 1, 8, SEQ), jnp.int32),
            pltpu.VMEM((K, 2, 1, SEQ, DIM), jnp.float32),
            pltpu.VMEM((K, 2, 1, 1, SEQ), jnp.int32),
            pltpu.SemaphoreType.DMA((K,)),
            pltpu.SemaphoreType.DMA((K,)),
        ],
    )

    prompts4, tok4 = pl.pallas_call(
        _body,
        grid_spec=grid_spec,
        out_shape=(
            jax.ShapeDtypeStruct((2, B, SEQ, DIM), jnp.float32),
            jax.ShapeDtypeStruct((2, B, 1, SEQ), jnp.int32),
        ),
        compiler_params=pltpu.CompilerParams(
            dimension_semantics=("arbitrary",)),
    )(cls_id, ctx_neg, ctx_pos, token_prefix_neg, token_prefix_pos,
      token_suffix_neg, token_suffix_pos,
      tokenized_prompts.reshape(2 * N_CLS // 8, 8, SEQ))

    return prompts4.reshape(2 * B, SEQ, DIM), tok4.reshape(2 * B, SEQ)


# speedup vs baseline: 6.9267x; 3.8766x over previous
import functools

import jax
import jax.numpy as jnp
from jax import lax
from jax.experimental import pallas as pl
from jax.experimental.pallas import tpu as pltpu
from jax.experimental.pallas import tpu_sc as plsc

N_CLS = 1000
DIM = 512
N_CTX = 16
SEQ = 77
SUF_L = SEQ - 1 - N_CTX
B = 256

NBUF = 6
ROWS_W = 16
TOK_ROWS = 3


def _sc_body(cls_hbm, pref_n, pref_p, ctx_n, ctx_p, suf_n, suf_p,
             out_hbm,
             cls_v, b0, b1, b2, b3, b4, b5,
             sg0, sg1, sg2, sg3, sg4, sg5, ss0, ss1, ss2, ss3, ss4, ss5):
    bufs = [b0, b1, b2, b3, b4, b5]
    sems_g = [sg0, sg1, sg2, sg3, sg4, sg5]
    sems_s = [ss0, ss1, ss2, ss3, ss4, ss5]
    nc = 2
    wid = lax.axis_index("s") * nc + lax.axis_index("c")
    half = wid // 16
    j = wid % 16

    pltpu.sync_copy(cls_hbm, cls_v)
    c16 = cls_v[pl.ds(j * ROWS_W, ROWS_W)]
    rowbase = 256 * half + ROWS_W * j

    def do_half(pref_t, ctx_t, suf_t):
        nstatic = 1 + N_CTX

        def suf_src(s):
            return suf_t.at[c16 + N_CLS * (s - 1 - N_CTX)]

        def src(s):
            if s == 0:
                return pref_t.at[c16]
            if s < nstatic:
                return ctx_t.at[c16 * N_CTX + (s - 1)]
            return suf_src(s)

        def fire(s, slot):
            return pltpu.async_copy(src(s), bufs[slot], sems_g[slot])

        def store(s, slot):
            return pltpu.async_copy(
                bufs[slot],
                out_hbm.at[pl.ds(DIM * s + rowbase, ROWS_W)],
                sems_s[slot])

        for s in range(NBUF):
            fire(s, s)
        for s in range(nstatic):
            slot = s % NBUF
            pltpu.make_async_copy(src(s), bufs[slot], sems_g[slot]).wait()
            store(s, slot).wait()
            fire(s + NBUF, slot)

        def chunk(c, _):
            for k in range(NBUF):
                s = nstatic + c * NBUF + k
                slot = (nstatic + k) % NBUF
                pltpu.make_async_copy(
                    suf_src(s), bufs[slot], sems_g[slot]).wait()
                store(s, slot).wait()

                @pl.when(s + NBUF < SEQ)
                def _(s=s, slot=slot):
                    pltpu.async_copy(
                        suf_src(s + NBUF), bufs[slot], sems_g[slot])
            return None

        lax.fori_loop(0, (SEQ - nstatic) // NBUF, chunk, None)

    @pl.when(half == 0)
    def _():
        do_half(pref_n, ctx_n, suf_n)

    @pl.when(half == 1)
    def _():
        do_half(pref_p, ctx_p, suf_p)


def _tok_body(cls_hbm, tokT, tokout_hbm, cls_v, tk_v, orv, sem):
    nc = 2
    wid = lax.axis_index("s") * nc + lax.axis_index("c")
    pltpu.sync_copy(cls_hbm, cls_v)
    for k in range(TOK_ROWS):
        st = wid * TOK_ROWS + k

        @pl.when(st < SEQ)
        def _(st=st):
            pltpu.sync_copy(tokT.at[st], tk_v)
            for i in range(32):
                ci = cls_v[pl.ds(16 * (i % 16), 16)]
                if i >= 16:
                    ci = ci + N_CLS
                orv[pl.ds(16 * i, 16)] = plsc.load_gather(tk_v, [ci])
            pltpu.sync_copy(orv, tokout_hbm.at[st])


def kernel(cls_id, ctx_pos, ctx_neg, token_prefix_pos, token_suffix_pos,
           token_prefix_neg, token_suffix_neg, tokenized_prompts):
    pref_n2 = token_prefix_neg.reshape(N_CLS, DIM)
    pref_p2 = token_prefix_pos.reshape(N_CLS, DIM)
    ctx_n2 = ctx_neg.reshape(N_CLS * N_CTX, DIM)
    ctx_p2 = ctx_pos.reshape(N_CLS * N_CTX, DIM)
    suf_n2 = token_suffix_neg.transpose(1, 0, 2).reshape(N_CLS * SUF_L, DIM)
    suf_p2 = token_suffix_pos.transpose(1, 0, 2).reshape(N_CLS * SUF_L, DIM)
    tokT = tokenized_prompts.transpose(1, 0)

    mesh = plsc.VectorSubcoreMesh(core_axis_name="c", subcore_axis_name="s")
    run = functools.partial(
        pl.kernel,
        mesh=mesh,
        out_type=jax.ShapeDtypeStruct((SEQ * 2 * B, DIM), jnp.float32),
        scratch_types=(
            [pltpu.VMEM((B,), jnp.int32)]
            + [pltpu.VMEM((ROWS_W, DIM), jnp.float32)] * NBUF
            + [pltpu.SemaphoreType.DMA] * (2 * NBUF)
        ),
    )(_sc_body)

    run_tok = functools.partial(
        pl.kernel,
        mesh=mesh,
        compiler_params=pltpu.CompilerParams(needs_layout_passes=False),
        out_type=jax.ShapeDtypeStruct((SEQ, 2 * B), jnp.int32),
        scratch_types=[
            pltpu.VMEM((B,), jnp.int32),
            pltpu.VMEM((2 * N_CLS,), jnp.int32),
            pltpu.VMEM((2 * B,), jnp.int32),
            pltpu.SemaphoreType.DMA,
        ],
    )(_tok_body)

    prompts_flat = run(
        cls_id, pref_n2, pref_p2, ctx_n2, ctx_p2, suf_n2, suf_p2)
    tokT_out = run_tok(cls_id, tokT)
    prompts = prompts_flat.reshape(SEQ, 2 * B, DIM).transpose(1, 0, 2)
    return prompts, tokT_out.transpose(1, 0)
